# Initial kernel scaffold; baseline (speedup 1.0000x reference)
import jax
import jax.numpy as jnp
from jax.experimental import pallas as pl


def kernel(inputs, table):
    raise NotImplementedError("write your pallas kernel here")


# speedup vs baseline: 2.5565x; 2.5565x over previous
import functools

import jax
import jax.numpy as jnp
from jax import lax
from jax.experimental import pallas as pl
from jax.experimental.pallas import tpu as pltpu
from jax.experimental.pallas import tpu_sc as plsc

B = 16384
L = 50
D = 32
NC = 2
NS = 16
NW = NC * NS
W = B // NW
C = 64
NCHUNK = W // C
IDX_PER_CHUNK = C * L
GSUB = 128
NGATHER = IDX_PER_CHUNK // GSUB


def _lane_bcast(vec, jvec):
  dnums = lax.GatherDimensionNumbers(
      offset_dims=(), collapsed_slice_dims=(0,), start_index_map=(0,))
  return lax.gather(vec, jvec[:, None], dnums, slice_sizes=(1,),
                    mode=lax.GatherScatterMode.PROMISE_IN_BOUNDS)


def _body(idx_hbm, table_hbm, out_hbm, idx_v, rows_v, nzf_v, inv_v, out_v,
          t0_v, sem):
  wid = lax.axis_index("s") * NC + lax.axis_index("c")
  row0 = wid * W

  pltpu.sync_copy(table_hbm.at[pl.ds(0, 1)], t0_v)
  t00 = t0_v[0, pl.ds(0, 16)]
  t01 = t0_v[0, pl.ds(16, 16)]
  lane = lax.iota(jnp.int32, 16)

  def chunk_body(c, _):
    base_row = row0 + c * C
    pltpu.sync_copy(idx_hbm.at[pl.ds(base_row * L, IDX_PER_CHUNK)], idx_v)

    def fire(i, _):
      pltpu.async_copy(table_hbm.at[idx_v.at[pl.ds(i * GSUB, GSUB)]],
                       rows_v.at[pl.ds(i * GSUB, GSUB)], sem)
      return 0
    lax.fori_loop(0, NGATHER, fire, 0)

    def count_group(g, _):
      def count_l(l, nz):
        vals = plsc.load_gather(idx_v, [lane * L + (g * 16 * L + l)])
        return nz + jnp.where(vals == 0, 1.0, 0.0)
      nzf = lax.fori_loop(0, L, count_l, jnp.zeros((16,), jnp.float32))
      cnt = jnp.float32(L) - nzf
      inv = jnp.where(cnt == 0.0, 0.0,
                      1.0 / jnp.where(cnt == 0.0, 1.0, cnt))
      nzf_v[pl.ds(g * 16, 16)] = nzf
      inv_v[pl.ds(g * 16, 16)] = inv
      return 0
    lax.fori_loop(0, C // 16, count_group, 0)

    pltpu.make_async_copy(table_hbm.at[pl.ds(0, IDX_PER_CHUNK)], rows_v,
                          sem).wait()

    def row_body(r, _):
      def acc_l(l, acc):
        a0, a1 = acc
        a0 = a0 + rows_v[r * L + l, pl.ds(0, 16)]
        a1 = a1 + rows_v[r * L + l, pl.ds(16, 16)]
        return (a0, a1)
      a0, a1 = lax.fori_loop(0, L, acc_l, (jnp.zeros((16,), jnp.float32),
                                           jnp.zeros((16,), jnp.float32)))
      g16 = (r // 16) * 16
      jvec = jnp.full((16,), r - g16, jnp.int32)
      nzb = _lane_bcast(nzf_v[pl.ds(g16, 16)], jvec)
      invb = _lane_bcast(inv_v[pl.ds(g16, 16)], jvec)
      out_v[r, pl.ds(0, 16)] = (a0 - nzb * t00) * invb
      out_v[r, pl.ds(16, 16)] = (a1 - nzb * t01) * invb
      return 0
    lax.fori_loop(0, C, row_body, 0)

    pltpu.sync_copy(out_v, out_hbm.at[pl.ds(base_row, C)])
    return 0

  lax.fori_loop(0, NCHUNK, chunk_body, 0)


@functools.partial(
    pl.kernel,
    out_type=jax.ShapeDtypeStruct((B, D), jnp.float32),
    mesh=plsc.VectorSubcoreMesh(core_axis_name="c", subcore_axis_name="s"),
    scratch_types=[
        pltpu.VMEM((IDX_PER_CHUNK,), jnp.int32),
        pltpu.VMEM((IDX_PER_CHUNK, D), jnp.float32),
        pltpu.VMEM((C,), jnp.float32),
        pltpu.VMEM((C,), jnp.float32),
        pltpu.VMEM((C, D), jnp.float32),
        pltpu.VMEM((1, D), jnp.float32),
        pltpu.SemaphoreType.DMA,
    ],
    compiler_params=pltpu.CompilerParams(needs_layout_passes=False,
                                         use_tc_tiling_on_sc=False),
)
def _masked_mean_sc(idx_hbm, table_hbm, out_hbm, idx_v, rows_v, nzf_v, inv_v,
                    out_v, t0_v, sem):
  _body(idx_hbm, table_hbm, out_hbm, idx_v, rows_v, nzf_v, inv_v, out_v,
        t0_v, sem)


def kernel(inputs, table):
  out = _masked_mean_sc(inputs.reshape(B * L), table)
  return out.reshape(B, 1, D)


# speedup vs baseline: 2.9554x; 1.1561x over previous
import functools

import jax
import jax.numpy as jnp
from jax import lax
from jax.experimental import pallas as pl
from jax.experimental.pallas import tpu as pltpu
from jax.experimental.pallas import tpu_sc as plsc

B = 16384
L = 50
D = 32
NC = 2
NS = 16
NW = NC * NS
W = B // NW
C = 32
CIDX = C * L
GSUB = 80
NG = CIDX // GSUB
NP = W // (2 * C)


def _lane_bcast(vec, jvec):
  dnums = lax.GatherDimensionNumbers(
      offset_dims=(), collapsed_slice_dims=(0,), start_index_map=(0,))
  return lax.gather(vec, jvec[:, None], dnums, slice_sizes=(1,),
                    mode=lax.GatherScatterMode.PROMISE_IN_BOUNDS)


def _body(idx_hbm, table_hbm, out_hbm, idx_a, idx_b, rows_a, rows_b,
          out_a, out_b, t0_v, sem_a, sem_b, isem_a, isem_b, osem_a, osem_b):
  wid = lax.axis_index("s") * NC + lax.axis_index("c")
  row0 = wid * W

  pltpu.sync_copy(table_hbm.at[pl.ds(0, 1)], t0_v)
  t00 = t0_v[0, pl.ds(0, 16)]
  t01 = t0_v[0, pl.ds(16, 16)]
  lane = lax.iota(jnp.int32, 16)

  def fire_gathers(idx_v, rows_v, sem):
    def fire(i, _):
      pltpu.async_copy(table_hbm.at[idx_v.at[pl.ds(i * GSUB, GSUB)]],
                       rows_v.at[pl.ds(i * GSUB, GSUB)], sem)
      return 0
    lax.fori_loop(0, NG, fire, 0)

  def wait_gathers(rows_v, sem):
    pltpu.make_async_copy(table_hbm.at[pl.ds(0, CIDX)], rows_v, sem).wait()

  def fire_idx(chunk, idx_v, isem):
    pltpu.async_copy(idx_hbm.at[pl.ds((row0 + chunk * C) * L, CIDX)],
                     idx_v, isem)

  def wait_idx(idx_v, isem):
    pltpu.make_async_copy(idx_hbm.at[pl.ds(0, CIDX)], idx_v, isem).wait()

  def counts(idx_v):
    res = []
    for g in range(C // 16):
      def count_l(l, nz):
        vals = plsc.load_gather(idx_v, [lane * L + (g * 16 * L + l)])
        return nz + jnp.where(vals == 0, 1.0, 0.0)
      nzf = lax.fori_loop(0, L, count_l, jnp.zeros((16,), jnp.float32),
                          unroll=10)
      cnt = jnp.float32(L) - nzf
      inv = jnp.where(cnt == 0.0, 0.0,
                      1.0 / jnp.where(cnt == 0.0, 1.0, cnt))
      res.append((nzf, inv))
    return res

  def accumulate(rows_v, out_v, cnts):
    def row_body(r, _):
      def acc_l(l, acc):
        a0, a1 = acc
        a0 = a0 + rows_v[r * L + l, pl.ds(0, 16)]
        a1 = a1 + rows_v[r * L + l, pl.ds(16, 16)]
        return (a0, a1)
      a0, a1 = lax.fori_loop(0, L, acc_l,
                             (jnp.zeros((16,), jnp.float32),
                              jnp.zeros((16,), jnp.float32)), unroll=10)
      g = r // 16
      jvec = jnp.full((16,), r - g * 16, jnp.int32)
      nz0, inv0 = cnts[0]
      nz1, inv1 = cnts[1]
      nzg = jnp.where(jnp.full((16,), g, jnp.int32) == 0, nz0, nz1)
      invg = jnp.where(jnp.full((16,), g, jnp.int32) == 0, inv0, inv1)
      nzb = _lane_bcast(nzg, jvec)
      invb = _lane_bcast(invg, jvec)
      out_v[r, pl.ds(0, 16)] = (a0 - nzb * t00) * invb
      out_v[r, pl.ds(16, 16)] = (a1 - nzb * t01) * invb
      return 0
    lax.fori_loop(0, C, row_body, 0)

  def fire_out(chunk, out_v, osem):
    pltpu.async_copy(out_v, out_hbm.at[pl.ds(row0 + chunk * C, C)], osem)

  def wait_out(out_v, osem):
    pltpu.make_async_copy(out_v, out_hbm.at[pl.ds(0, C)], osem).wait()

  pltpu.sync_copy(idx_hbm.at[pl.ds(row0 * L, CIDX)], idx_a)
  fire_gathers(idx_a, rows_a, sem_a)
  pltpu.sync_copy(idx_hbm.at[pl.ds((row0 + C) * L, CIDX)], idx_b)

  def pair_body(p, _):
    e = 2 * p
    o = 2 * p + 1

    @pl.when(p > 0)
    def _():
      wait_idx(idx_b, isem_b)
    fire_gathers(idx_b, rows_b, sem_b)

    cnts_e = counts(idx_a)
    wait_gathers(rows_a, sem_a)

    @pl.when(p > 0)
    def _():
      wait_out(out_a, osem_a)
    accumulate(rows_a, out_a, cnts_e)
    fire_out(e, out_a, osem_a)

    @pl.when(p < NP - 1)
    def _():
      fire_idx(e + 2, idx_a, isem_a)
      wait_idx(idx_a, isem_a)
      fire_gathers(idx_a, rows_a, sem_a)

    cnts_o = counts(idx_b)
    wait_gathers(rows_b, sem_b)

    @pl.when(p > 0)
    def _():
      wait_out(out_b, osem_b)
    accumulate(rows_b, out_b, cnts_o)
    fire_out(o, out_b, osem_b)

    @pl.when(p < NP - 1)
    def _():
      fire_idx(o + 2, idx_b, isem_b)
    return 0

  lax.fori_loop(0, NP, pair_body, 0)
  wait_out(out_a, osem_a)
  wait_out(out_b, osem_b)


@functools.partial(
    pl.kernel,
    out_type=jax.ShapeDtypeStruct((B, D), jnp.float32),
    mesh=plsc.VectorSubcoreMesh(core_axis_name="c", subcore_axis_name="s"),
    scratch_types=[
        pltpu.VMEM((CIDX,), jnp.int32),
        pltpu.VMEM((CIDX,), jnp.int32),
        pltpu.VMEM((CIDX, D), jnp.float32),
        pltpu.VMEM((CIDX, D), jnp.float32),
        pltpu.VMEM((C, D), jnp.float32),
        pltpu.VMEM((C, D), jnp.float32),
        pltpu.VMEM((1, D), jnp.float32),
        pltpu.SemaphoreType.DMA,
        pltpu.SemaphoreType.DMA,
        pltpu.SemaphoreType.DMA,
        pltpu.SemaphoreType.DMA,
        pltpu.SemaphoreType.DMA,
        pltpu.SemaphoreType.DMA,
    ],
    compiler_params=pltpu.CompilerParams(needs_layout_passes=False,
                                         use_tc_tiling_on_sc=False),
)
def _masked_mean_sc(idx_hbm, table_hbm, out_hbm, idx_a, idx_b, rows_a,
                    rows_b, out_a, out_b, t0_v, sem_a, sem_b, isem_a,
                    isem_b, osem_a, osem_b):
  _body(idx_hbm, table_hbm, out_hbm, idx_a, idx_b, rows_a, rows_b,
        out_a, out_b, t0_v, sem_a, sem_b, isem_a, isem_b, osem_a, osem_b)


def kernel(inputs, table):
  out = _masked_mean_sc(inputs.reshape(B * L), table)
  return out.reshape(B, 1, D)
